# baseline (speedup 1.0000x reference)
import jax
import jax.numpy as jnp
from jax.experimental import pallas as pl
from jax.experimental.pallas import tpu as pltpu

_CHUNKS = 10


def _copy_body(x_ref, e_ref, x_out, e_out, sem):
    nx = x_ref.shape[0] // _CHUNKS
    ne = e_ref.shape[0] // _CHUNKS
    copies = []
    for i in range(_CHUNKS):
        copies.append(pltpu.make_async_copy(
            x_ref.at[pl.ds(i * nx, nx), :], x_out.at[pl.ds(i * nx, nx), :],
            sem.at[i]))
        copies.append(pltpu.make_async_copy(
            e_ref.at[pl.ds(i * ne, ne), :], e_out.at[pl.ds(i * ne, ne), :],
            sem.at[_CHUNKS + i]))
    for c in copies:
        c.start()
    for c in copies:
        c.wait()


def kernel(x, x_lstm, encoded_z_gnss, edge_index, edge_attr):
    x_out, e_out = pl.pallas_call(
        _copy_body,
        out_shape=(
            jax.ShapeDtypeStruct(x.shape, x.dtype),
            jax.ShapeDtypeStruct(edge_attr.shape, edge_attr.dtype),
        ),
        in_specs=[
            pl.BlockSpec(memory_space=pl.ANY),
            pl.BlockSpec(memory_space=pl.ANY),
        ],
        out_specs=(
            pl.BlockSpec(memory_space=pl.ANY),
            pl.BlockSpec(memory_space=pl.ANY),
        ),
        scratch_shapes=[pltpu.SemaphoreType.DMA((2 * _CHUNKS,))],
    )(x, edge_attr)
    return (x_out, e_out)
